# baseline (speedup 1.0000x reference)
import dataclasses
import functools

import jax
import jax.numpy as jnp
import numpy as _np
from jax import lax
from jax.experimental import pallas as pl
from jax.experimental.pallas import tpu as pltpu
from jax.experimental.pallas import tpu_sc as plsc

N = 10000
E = 160000
D = 256
H = 256

NP = 10240
ROWS_BLK = 2048
N_BLOCKS = NP // ROWS_BLK

NTILES = 32
CHUNK = 80
NCHUNK = 64
EP_TILE = NCHUNK * CHUNK
EP = NTILES * EP_TILE
ZROWS = NP // 16

_mesh = plsc.VectorSubcoreMesh(core_axis_name="c", subcore_axis_name="s")

_PERM = _np.concatenate(
    [_np.concatenate([_np.arange(32 * k, 32 * k + 32, 2),
                      _np.arange(32 * k + 1, 32 * k + 32, 2)])
     for k in range(4)])

_sc_params = pltpu.CompilerParams()
if "needs_layout_passes" in pltpu.CompilerParams.__dataclass_fields__:
    _sc_params = dataclasses.replace(_sc_params, needs_layout_passes=False)
_sc_params_s2 = dataclasses.replace(_sc_params, use_tc_tiling_on_sc=False)


def _t1_body(x_ref, w_ref, a_ref, hs_ref, asd_ref):
    h = jnp.dot(x_ref[...], w_ref[...], preferred_element_type=jnp.float32)
    hs_ref[0] = h[:, :128].astype(jnp.bfloat16)
    hs_ref[1] = h[:, 128:].astype(jnp.bfloat16)
    a2 = jnp.dot(h, a_ref[...], preferred_element_type=jnp.float32)
    asd_ref[...] = a2.T


def _t1(x_pad, W, A):
    return pl.pallas_call(
        _t1_body,
        grid=(N_BLOCKS,),
        in_specs=[
            pl.BlockSpec((ROWS_BLK, D), lambda i: (i, 0)),
            pl.BlockSpec((D, H), lambda i: (0, 0)),
            pl.BlockSpec((H, 2), lambda i: (0, 0)),
        ],
        out_specs=[
            pl.BlockSpec((2, ROWS_BLK, 128), lambda i: (0, i, 0)),
            pl.BlockSpec((2, ROWS_BLK), lambda i: (0, i)),
        ],
        out_shape=[
            jax.ShapeDtypeStruct((2, NP, 128), jnp.bfloat16),
            jax.ShapeDtypeStruct((2, NP), jnp.float32),
        ],
    )(x_pad, W, A)


@functools.partial(
    pl.kernel,
    out_type=[
        jax.ShapeDtypeStruct((NTILES, NCHUNK, CHUNK), jnp.float32),
        jax.ShapeDtypeStruct((2, NP), jnp.float32),
    ],
    mesh=_mesh,
    compiler_params=_sc_params,
    scratch_types=[
        pltpu.VMEM((NP,), jnp.float32),
        pltpu.VMEM((NP,), jnp.float32),
        pltpu.VMEM((NCHUNK, CHUNK), jnp.int32),
        pltpu.VMEM((NCHUNK, CHUNK), jnp.int32),
        pltpu.VMEM((NCHUNK, CHUNK), jnp.float32),
        pltpu.VMEM((ZROWS,), jnp.float32),
        pltpu.VMEM_SHARED((NP,), jnp.float32),
    ],
)
def _s1(asd_hbm, srcm_hbm, dstm_hbm, exm_hbm, spart_hbm,
        as_v, ad_v, src_v, dst_v, ex_v, z_v, s_sh):
    c = lax.axis_index("c")
    sid = lax.axis_index("s")
    row = sid * 2 + c

    @pl.loop(0, ZROWS // 16)
    def _(i):
        z_v[pl.ds(i * 16, 16)] = jnp.zeros((16,), jnp.float32)

    pltpu.sync_copy(z_v, s_sh.at[pl.ds(sid * ZROWS, ZROWS)])
    pltpu.sync_copy(asd_hbm.at[0], as_v)
    pltpu.sync_copy(asd_hbm.at[1], ad_v)
    pltpu.sync_copy(srcm_hbm.at[row], src_v)
    pltpu.sync_copy(dstm_hbm.at[row], dst_v)
    plsc.subcore_barrier()

    @pl.loop(0, NCHUNK)
    def _(j):
        @pl.loop(0, CHUNK // 16)
        def _(k):
            si = src_v[j, pl.ds(k * 16, 16)]
            di = dst_v[j, pl.ds(k * 16, 16)]
            e = plsc.load_gather(as_v, [si]) + plsc.load_gather(ad_v, [di])
            e = jnp.where(e > 0, e, e * jnp.float32(0.2))
            ex_v[j, pl.ds(k * 16, 16)] = jnp.exp(e)

        pltpu.sync_copy(ex_v.at[j], s_sh.at[dst_v.at[j]], add=True)

    pltpu.sync_copy(ex_v, exm_hbm.at[row])
    plsc.subcore_barrier()
    pltpu.sync_copy(s_sh.at[pl.ds(sid * ZROWS, ZROWS)],
                    spart_hbm.at[c, pl.ds(sid * ZROWS, ZROWS)])


@functools.partial(
    pl.kernel,
    out_type=jax.ShapeDtypeStruct((2, NP, 128), jnp.float32),
    mesh=_mesh,
    compiler_params=_sc_params_s2,
    scratch_types=[
        pltpu.VMEM((NCHUNK, CHUNK), jnp.int32),
        pltpu.VMEM((NCHUNK, CHUNK), jnp.int32),
        pltpu.VMEM((NCHUNK, CHUNK), jnp.float32),
        pltpu.VMEM((CHUNK, 64), jnp.int32),
        pltpu.VMEM((CHUNK, 64), jnp.int32),
        pltpu.VMEM((CHUNK, 128), jnp.float32),
        pltpu.VMEM((CHUNK, 128), jnp.float32),
        pltpu.VMEM_SHARED((NP, 128), jnp.float32),
        pltpu.SemaphoreType.DMA,
        pltpu.SemaphoreType.DMA,
        pltpu.SemaphoreType.DMA,
        pltpu.SemaphoreType.DMA,
    ],
)
def _s2(hs_hbm, srcm_hbm, dstm_hbm, exm_hbm, outs_hbm,
        src_v, dst_v, ex_v, rowsb_a, rowsb_b, rows_a, rows_b, out_sh,
        gsa, gsb, ssa, ssb):
    c = lax.axis_index("c")
    sid = lax.axis_index("s")

    @pl.loop(0, CHUNK)
    def _(m):
        @pl.loop(0, 8)
        def _(k):
            rows_a[m, pl.ds(k * 16, 16)] = jnp.zeros((16,), jnp.float32)

    @pl.loop(0, ZROWS // CHUNK)
    def _(i):
        pltpu.sync_copy(rows_a,
                        out_sh.at[pl.ds(sid * ZROWS + i * CHUNK, CHUNK)])

    plsc.subcore_barrier()

    def gstart(bbuf, sem, j):
        pltpu.async_copy(hs_hbm.at[c].at[src_v.at[j]], bbuf, sem)

    def gwait(bbuf, sem, j):
        pltpu.make_async_copy(hs_hbm.at[c].at[src_v.at[j]], bbuf, sem).wait()

    def sstart(fbuf, sem, j):
        pltpu.async_copy(fbuf, out_sh.at[dst_v.at[j]], sem, add=True)

    def swait(fbuf, sem, j):
        pltpu.make_async_copy(fbuf, out_sh.at[dst_v.at[j]], sem).wait()

    def scale(bbuf, fbuf, j):
        @pl.loop(0, CHUNK, unroll=4)
        def _(m):
            g = plsc.load_gather(
                ex_v, [jnp.full((16,), j, jnp.int32),
                       jnp.full((16,), m, jnp.int32)])
            for k in range(4):
                c32 = plsc.bitcast(bbuf[m, pl.ds(k * 16, 16)], jnp.bfloat16)
                a, b = plsc.unpack(c32, format=plsc.PackFormat.INTERLEAVED)
                fbuf[m, pl.ds(k * 32, 16)] = a * g
                fbuf[m, pl.ds(k * 32 + 16, 16)] = b * g

    for r in range(2):
        row = sid * 2 + r
        pltpu.sync_copy(srcm_hbm.at[row], src_v)
        pltpu.sync_copy(dstm_hbm.at[row], dst_v)
        pltpu.sync_copy(exm_hbm.at[row], ex_v)

        gstart(rowsb_a, gsa, 0)

        @pl.loop(0, NCHUNK // 2)
        def _(jj):
            j = jj * 2

            @pl.when(jj > 0)
            def _():
                swait(rows_b, ssb, j - 1)

            gstart(rowsb_b, gsb, j + 1)
            gwait(rowsb_a, gsa, j)
            scale(rowsb_a, rows_a, j)
            sstart(rows_a, ssa, j)

            gwait(rowsb_b, gsb, j + 1)
            scale(rowsb_b, rows_b, j + 1)
            sstart(rows_b, ssb, j + 1)

            @pl.when(jj < NCHUNK // 2 - 1)
            def _():
                swait(rows_a, ssa, j)
                gstart(rowsb_a, gsa, j + 2)

        swait(rows_a, ssa, NCHUNK - 2)
        swait(rows_b, ssb, NCHUNK - 1)

    plsc.subcore_barrier()
    pltpu.sync_copy(out_sh.at[pl.ds(sid * ZROWS, ZROWS)],
                    outs_hbm.at[c, pl.ds(sid * ZROWS, ZROWS)])


def _t2_body(o0_ref, o1_ref, sp_ref, b_ref, f0_ref, f1_ref, bfc2_ref, pq_ref):
    invs = 1.0 / (sp_ref[0] + sp_ref[1] + jnp.float32(1e-16))
    v0 = o0_ref[0] * invs[:, None] + b_ref[0, :128]
    v1 = o1_ref[0] * invs[:, None] + b_ref[0, 128:]
    e0 = jnp.where(v0 > 0, v0, jnp.exp(v0) - 1.0)
    e1 = jnp.where(v1 > 0, v1, jnp.exp(v1) - 1.0)
    pq = (jnp.dot(e0, f0_ref[...], preferred_element_type=jnp.float32)
          + jnp.dot(e1, f1_ref[...], preferred_element_type=jnp.float32))
    pq_ref[...] = pq.T + bfc2_ref[...]


def _t2(outs, spart, bias2d, F0, F1, bfc2):
    return pl.pallas_call(
        _t2_body,
        grid=(N_BLOCKS,),
        in_specs=[
            pl.BlockSpec((1, ROWS_BLK, 128), lambda i: (0, i, 0)),
            pl.BlockSpec((1, ROWS_BLK, 128), lambda i: (1, i, 0)),
            pl.BlockSpec((2, ROWS_BLK), lambda i: (0, i)),
            pl.BlockSpec((1, D), lambda i: (0, 0)),
            pl.BlockSpec((128, 2), lambda i: (0, 0)),
            pl.BlockSpec((128, 2), lambda i: (0, 0)),
            pl.BlockSpec((2, 1), lambda i: (0, 0)),
        ],
        out_specs=pl.BlockSpec((2, ROWS_BLK), lambda i: (0, i)),
        out_shape=jax.ShapeDtypeStruct((2, NP), jnp.float32),
    )(outs, outs, spart, bias2d, F0, F1, bfc2)


@functools.partial(
    pl.kernel,
    out_type=jax.ShapeDtypeStruct((NTILES, NCHUNK, CHUNK), jnp.float32),
    mesh=_mesh,
    compiler_params=_sc_params,
    scratch_types=[
        pltpu.VMEM((NP,), jnp.float32),
        pltpu.VMEM((NP,), jnp.float32),
        pltpu.VMEM((NCHUNK, CHUNK), jnp.int32),
        pltpu.VMEM((NCHUNK, CHUNK), jnp.int32),
        pltpu.VMEM((NCHUNK, CHUNK), jnp.float32),
    ],
)
def _s3(pq_hbm, srcm_hbm, dstm_hbm, scoresm_hbm,
        p_v, q_v, src_v, dst_v, sc_v):
    c = lax.axis_index("c")
    sid = lax.axis_index("s")
    row = sid * 2 + c
    pltpu.sync_copy(pq_hbm.at[0], p_v)
    pltpu.sync_copy(pq_hbm.at[1], q_v)
    pltpu.sync_copy(srcm_hbm.at[row], src_v)
    pltpu.sync_copy(dstm_hbm.at[row], dst_v)

    @pl.loop(0, NCHUNK)
    def _(j):
        @pl.loop(0, CHUNK // 16)
        def _(k):
            si = src_v[j, pl.ds(k * 16, 16)]
            di = dst_v[j, pl.ds(k * 16, 16)]
            sc_v[j, pl.ds(k * 16, 16)] = (
                plsc.load_gather(p_v, [si]) + plsc.load_gather(q_v, [di]))

    pltpu.sync_copy(sc_v, scoresm_hbm.at[row])


def kernel(x, edge_index, W, att_src, att_dst, bias, Wfc, bfc):
    x_pad = jnp.concatenate(
        [x, jnp.zeros((NP - N, D), jnp.float32)], axis=0)
    A = jnp.stack([att_src, att_dst], axis=1)

    src = edge_index[0]
    dst = edge_index[1]
    srcm = jnp.concatenate(
        [src, jnp.zeros((EP - E,), jnp.int32)]).reshape(NTILES, NCHUNK, CHUNK)
    dstm = jnp.concatenate(
        [dst, jnp.full((EP - E,), N, jnp.int32)]).reshape(NTILES, NCHUNK, CHUNK)

    hs, asd = _t1(x_pad, W, A)
    hs32 = jax.lax.bitcast_convert_type(
        hs.reshape(2, NP, 64, 2), jnp.int32)
    exm, spart = _s1(asd, srcm, dstm)
    outs = _s2(hs32, srcm, dstm, exm)

    bias2d = jnp.concatenate(
        [bias[:128][_PERM], bias[128:][_PERM]]).reshape(1, D)
    F0 = jnp.stack([Wfc[:128, 0][_PERM], Wfc[H:H + 128, 0][_PERM]], axis=1)
    F1 = jnp.stack([Wfc[128:H, 0][_PERM], Wfc[H + 128:, 0][_PERM]], axis=1)
    bfc2 = jnp.stack([bfc, jnp.zeros((1,), jnp.float32)])
    pq = _t2(outs, spart, bias2d, F0, F1, bfc2)

    scoresm = _s3(pq, srcm, dstm)
    return scoresm.reshape(EP)[:E]


# speedup vs baseline: 1.8156x; 1.8156x over previous
import dataclasses
import functools

import jax
import jax.numpy as jnp
from jax import lax
from jax.experimental import pallas as pl
from jax.experimental.pallas import tpu as pltpu
from jax.experimental.pallas import tpu_sc as plsc

N = 10000
E = 160000
D = 256
H = 256

NP = 10240
ROWS_BLK = 2048
N_BLOCKS = NP // ROWS_BLK

NTILES = 32
CHUNK = 128
NCHUNK = 40
EP_TILE = NCHUNK * CHUNK
EP = NTILES * EP_TILE
ZROWS = NP // 16

_mesh = plsc.VectorSubcoreMesh(core_axis_name="c", subcore_axis_name="s")

_sc_params = pltpu.CompilerParams()
if "needs_layout_passes" in pltpu.CompilerParams.__dataclass_fields__:
    _sc_params = dataclasses.replace(_sc_params, needs_layout_passes=False)


def _t1_body(x_ref, w_ref, a_ref, hs_ref, asd_ref):
    h = jnp.dot(x_ref[...], w_ref[...], preferred_element_type=jnp.float32)
    hs_ref[0] = h[:, :128]
    hs_ref[1] = h[:, 128:]
    a2 = jnp.dot(h, a_ref[...], preferred_element_type=jnp.float32)
    asd_ref[...] = a2.T


def _t1(x_pad, W, A):
    return pl.pallas_call(
        _t1_body,
        grid=(N_BLOCKS,),
        in_specs=[
            pl.BlockSpec((ROWS_BLK, D), lambda i: (i, 0)),
            pl.BlockSpec((D, H), lambda i: (0, 0)),
            pl.BlockSpec((H, 2), lambda i: (0, 0)),
        ],
        out_specs=[
            pl.BlockSpec((2, ROWS_BLK, 128), lambda i: (0, i, 0)),
            pl.BlockSpec((2, ROWS_BLK), lambda i: (0, i)),
        ],
        out_shape=[
            jax.ShapeDtypeStruct((2, NP, 128), jnp.float32),
            jax.ShapeDtypeStruct((2, NP), jnp.float32),
        ],
    )(x_pad, W, A)


@functools.partial(
    pl.kernel,
    out_type=[
        jax.ShapeDtypeStruct((NTILES, NCHUNK, CHUNK), jnp.float32),
        jax.ShapeDtypeStruct((2, NP), jnp.float32),
    ],
    mesh=_mesh,
    compiler_params=_sc_params,
    scratch_types=[
        pltpu.VMEM((NP,), jnp.float32),
        pltpu.VMEM((NP,), jnp.float32),
        pltpu.VMEM((NCHUNK, CHUNK), jnp.int32),
        pltpu.VMEM((NCHUNK, CHUNK), jnp.int32),
        pltpu.VMEM((NCHUNK, CHUNK), jnp.float32),
        pltpu.VMEM((ZROWS,), jnp.float32),
        pltpu.VMEM_SHARED((NP,), jnp.float32),
    ],
)
def _s1(asd_hbm, srcm_hbm, dstm_hbm, exm_hbm, spart_hbm,
        as_v, ad_v, src_v, dst_v, ex_v, z_v, s_sh):
    c = lax.axis_index("c")
    sid = lax.axis_index("s")
    row = sid * 2 + c

    @pl.loop(0, ZROWS // 16)
    def _(i):
        z_v[pl.ds(i * 16, 16)] = jnp.zeros((16,), jnp.float32)

    pltpu.sync_copy(z_v, s_sh.at[pl.ds(sid * ZROWS, ZROWS)])
    pltpu.sync_copy(asd_hbm.at[0], as_v)
    pltpu.sync_copy(asd_hbm.at[1], ad_v)
    pltpu.sync_copy(srcm_hbm.at[row], src_v)
    pltpu.sync_copy(dstm_hbm.at[row], dst_v)
    plsc.subcore_barrier()

    @pl.loop(0, NCHUNK)
    def _(j):
        @pl.loop(0, CHUNK // 16)
        def _(k):
            si = src_v[j, pl.ds(k * 16, 16)]
            di = dst_v[j, pl.ds(k * 16, 16)]
            e = plsc.load_gather(as_v, [si]) + plsc.load_gather(ad_v, [di])
            e = jnp.where(e > 0, e, e * jnp.float32(0.2))
            ex_v[j, pl.ds(k * 16, 16)] = jnp.exp(e)

        pltpu.sync_copy(ex_v.at[j], s_sh.at[dst_v.at[j]], add=True)

    pltpu.sync_copy(ex_v, exm_hbm.at[row])
    plsc.subcore_barrier()
    pltpu.sync_copy(s_sh.at[pl.ds(sid * ZROWS, ZROWS)],
                    spart_hbm.at[c, pl.ds(sid * ZROWS, ZROWS)])


@functools.partial(
    pl.kernel,
    out_type=jax.ShapeDtypeStruct((2, NP, 128), jnp.float32),
    mesh=_mesh,
    compiler_params=_sc_params,
    scratch_types=[
        pltpu.VMEM((NCHUNK, CHUNK), jnp.int32),
        pltpu.VMEM((NCHUNK, CHUNK), jnp.int32),
        pltpu.VMEM((NCHUNK, CHUNK), jnp.float32),
        pltpu.VMEM((CHUNK, 128), jnp.float32),
        pltpu.VMEM((CHUNK, 128), jnp.float32),
        pltpu.VMEM_SHARED((NP, 128), jnp.float32),
        pltpu.SemaphoreType.DMA,
        pltpu.SemaphoreType.DMA,
        pltpu.SemaphoreType.DMA,
        pltpu.SemaphoreType.DMA,
    ],
)
def _s2(hs_hbm, srcm_hbm, dstm_hbm, exm_hbm, outs_hbm,
        src_v, dst_v, ex_v, rows_a, rows_b, out_sh,
        gsa, gsb, ssa, ssb):
    c = lax.axis_index("c")
    sid = lax.axis_index("s")

    @pl.loop(0, CHUNK)
    def _(m):
        @pl.loop(0, 8)
        def _(k):
            rows_a[m, pl.ds(k * 16, 16)] = jnp.zeros((16,), jnp.float32)

    @pl.loop(0, ZROWS // CHUNK)
    def _(i):
        pltpu.sync_copy(rows_a,
                        out_sh.at[pl.ds(sid * ZROWS + i * CHUNK, CHUNK)])

    plsc.subcore_barrier()

    def gstart(buf, sem, j):
        pltpu.async_copy(hs_hbm.at[c].at[src_v.at[j]], buf, sem)

    def gwait(buf, sem, j):
        pltpu.make_async_copy(hs_hbm.at[c].at[src_v.at[j]], buf, sem).wait()

    def sstart(buf, sem, j):
        pltpu.async_copy(buf, out_sh.at[dst_v.at[j]], sem, add=True)

    def swait(buf, sem, j):
        pltpu.make_async_copy(buf, out_sh.at[dst_v.at[j]], sem).wait()

    def scale(buf, j):
        @pl.loop(0, CHUNK, unroll=4)
        def _(m):
            g = plsc.load_gather(
                ex_v, [jnp.full((16,), j, jnp.int32),
                       jnp.full((16,), m, jnp.int32)])
            for k in range(8):
                buf[m, pl.ds(k * 16, 16)] = buf[m, pl.ds(k * 16, 16)] * g

    for r in range(2):
        row = sid * 2 + r
        pltpu.sync_copy(srcm_hbm.at[row], src_v)
        pltpu.sync_copy(dstm_hbm.at[row], dst_v)
        pltpu.sync_copy(exm_hbm.at[row], ex_v)

        gstart(rows_a, gsa, 0)

        @pl.loop(0, NCHUNK // 2)
        def _(jj):
            j = jj * 2

            @pl.when(jj > 0)
            def _():
                swait(rows_b, ssb, j - 1)

            gstart(rows_b, gsb, j + 1)
            gwait(rows_a, gsa, j)
            scale(rows_a, j)
            sstart(rows_a, ssa, j)

            gwait(rows_b, gsb, j + 1)
            scale(rows_b, j + 1)
            sstart(rows_b, ssb, j + 1)

            @pl.when(jj < NCHUNK // 2 - 1)
            def _():
                swait(rows_a, ssa, j)
                gstart(rows_a, gsa, j + 2)

        swait(rows_a, ssa, NCHUNK - 2)
        swait(rows_b, ssb, NCHUNK - 1)

    plsc.subcore_barrier()
    pltpu.sync_copy(out_sh.at[pl.ds(sid * ZROWS, ZROWS)],
                    outs_hbm.at[c, pl.ds(sid * ZROWS, ZROWS)])


def _t2_body(o0_ref, o1_ref, sp_ref, b_ref, f0_ref, f1_ref, bfc2_ref, pq_ref):
    invs = 1.0 / (sp_ref[0] + sp_ref[1] + jnp.float32(1e-16))
    v0 = o0_ref[0] * invs[:, None] + b_ref[0, :128]
    v1 = o1_ref[0] * invs[:, None] + b_ref[0, 128:]
    e0 = jnp.where(v0 > 0, v0, jnp.exp(v0) - 1.0)
    e1 = jnp.where(v1 > 0, v1, jnp.exp(v1) - 1.0)
    pq = (jnp.dot(e0, f0_ref[...], preferred_element_type=jnp.float32)
          + jnp.dot(e1, f1_ref[...], preferred_element_type=jnp.float32))
    pq_ref[...] = pq.T + bfc2_ref[...]


def _t2(outs, spart, bias2d, F0, F1, bfc2):
    return pl.pallas_call(
        _t2_body,
        grid=(N_BLOCKS,),
        in_specs=[
            pl.BlockSpec((1, ROWS_BLK, 128), lambda i: (0, i, 0)),
            pl.BlockSpec((1, ROWS_BLK, 128), lambda i: (1, i, 0)),
            pl.BlockSpec((2, ROWS_BLK), lambda i: (0, i)),
            pl.BlockSpec((1, D), lambda i: (0, 0)),
            pl.BlockSpec((128, 2), lambda i: (0, 0)),
            pl.BlockSpec((128, 2), lambda i: (0, 0)),
            pl.BlockSpec((2, 1), lambda i: (0, 0)),
        ],
        out_specs=pl.BlockSpec((2, ROWS_BLK), lambda i: (0, i)),
        out_shape=jax.ShapeDtypeStruct((2, NP), jnp.float32),
    )(outs, outs, spart, bias2d, F0, F1, bfc2)


@functools.partial(
    pl.kernel,
    out_type=jax.ShapeDtypeStruct((NTILES, NCHUNK, CHUNK), jnp.float32),
    mesh=_mesh,
    compiler_params=_sc_params,
    scratch_types=[
        pltpu.VMEM((NP,), jnp.float32),
        pltpu.VMEM((NP,), jnp.float32),
        pltpu.VMEM((NCHUNK, CHUNK), jnp.int32),
        pltpu.VMEM((NCHUNK, CHUNK), jnp.int32),
        pltpu.VMEM((NCHUNK, CHUNK), jnp.float32),
    ],
)
def _s3(pq_hbm, srcm_hbm, dstm_hbm, scoresm_hbm,
        p_v, q_v, src_v, dst_v, sc_v):
    c = lax.axis_index("c")
    sid = lax.axis_index("s")
    row = sid * 2 + c
    pltpu.sync_copy(pq_hbm.at[0], p_v)
    pltpu.sync_copy(pq_hbm.at[1], q_v)
    pltpu.sync_copy(srcm_hbm.at[row], src_v)
    pltpu.sync_copy(dstm_hbm.at[row], dst_v)

    @pl.loop(0, NCHUNK)
    def _(j):
        @pl.loop(0, CHUNK // 16)
        def _(k):
            si = src_v[j, pl.ds(k * 16, 16)]
            di = dst_v[j, pl.ds(k * 16, 16)]
            sc_v[j, pl.ds(k * 16, 16)] = (
                plsc.load_gather(p_v, [si]) + plsc.load_gather(q_v, [di]))

    pltpu.sync_copy(sc_v, scoresm_hbm.at[row])


def kernel(x, edge_index, W, att_src, att_dst, bias, Wfc, bfc):
    x_pad = jnp.concatenate(
        [x, jnp.zeros((NP - N, D), jnp.float32)], axis=0)
    A = jnp.stack([att_src, att_dst], axis=1)

    src = edge_index[0]
    dst = edge_index[1]
    pad_i = jnp.arange(EP - E, dtype=jnp.int32)
    srcm = jnp.concatenate(
        [src, pad_i % N]).reshape(NTILES, NCHUNK, CHUNK)
    dstm = jnp.concatenate(
        [dst, N + pad_i % (NP - N)]).reshape(NTILES, NCHUNK, CHUNK)

    hs, asd = _t1(x_pad, W, A)
    exm, spart = _s1(asd, srcm, dstm)
    outs = _s2(hs, srcm, dstm, exm)

    bias2d = bias.reshape(1, D)
    F0 = jnp.stack([Wfc[:128, 0], Wfc[H:H + 128, 0]], axis=1)
    F1 = jnp.stack([Wfc[128:H, 0], Wfc[H + 128:, 0]], axis=1)
    bfc2 = jnp.stack([bfc, jnp.zeros((1,), jnp.float32)])
    pq = _t2(outs, spart, bias2d, F0, F1, bfc2)

    scoresm = _s3(pq, srcm, dstm)
    return scoresm.reshape(EP)[:E]
